# baseline (speedup 1.0000x reference)
import functools

import jax
import jax.numpy as jnp
from jax import lax
from jax.experimental import pallas as pl
from jax.experimental.pallas import tpu as pltpu
from jax.experimental.pallas import tpu_sc as plsc

B = 16384
D = 64
DP = 128
V = 1000000
NBLK = 7813
VP = NBLK * 128
NNEG = 10
NW = 32
BPW = B // NW
CHUNK = 64
NCHUNK = BPW // CHUNK
LANES = 16
GROUPS = CHUNK // LANES
TRIPS = 123


def _sc_convert(tt):
    mesh = plsc.VectorSubcoreMesh(core_axis_name="c", subcore_axis_name="s")

    @functools.partial(
        pl.kernel,
        mesh=mesh,
        out_type=jax.ShapeDtypeStruct((VP, DP), jnp.float32),
        scratch_types=[
            pltpu.VMEM((D, 128), jnp.float32),
            pltpu.VMEM((D, 128), jnp.float32),
            pltpu.VMEM((128, DP), jnp.float32),
            pltpu.VMEM((128, DP), jnp.float32),
            pltpu.SemaphoreType.DMA,
            pltpu.SemaphoreType.DMA,
            pltpu.SemaphoreType.DMA,
            pltpu.SemaphoreType.DMA,
        ],
        compiler_params=pltpu.CompilerParams(
            needs_layout_passes=False, use_tc_tiling_on_sc=True,
            disable_bounds_checks=True),
    )
    def conv_kernel(tt_hbm, conv_hbm, inb0, inb1, outb0, outb1,
                    rsem0, rsem1, wsem0, wsem1):
        wid = lax.axis_index("s") * 2 + lax.axis_index("c")
        inbs = (inb0, inb1)
        outbs = (outb0, outb1)
        rsems = (rsem0, rsem1)
        wsems = (wsem0, wsem1)
        lanes = lax.iota(jnp.int32, LANES)

        zeros = jnp.zeros((LANES,), jnp.float32)
        def z_body(r, zcarry):
            for h in range(2):
                for j in range(D // LANES):
                    outbs[h][r, pl.ds(D + j * LANES, LANES)] = zeros
            return zcarry
        lax.fori_loop(0, 128, z_body, 0)

        for h in range(2):
            bid0 = wid + h * NW
            pltpu.async_copy(tt_hbm.at[:, pl.ds(bid0 * 128, 128)],
                             inbs[h], rsems[h])

        def trip(t, carry):
            for h in range(2):
                bid = wid + (2 * t + h) * NW
                nbid = bid + 2 * NW

                @pl.when(bid < NBLK)
                def _process():
                    @pl.when(2 * t + h >= 2)
                    def _w():
                        pltpu.make_async_copy(
                            outbs[h], conv_hbm.at[pl.ds(0, 128), :],
                            wsems[h]).wait()
                    pltpu.make_async_copy(
                        tt_hbm.at[:, pl.ds(bid * 128, 128)], inbs[h],
                        rsems[h]).wait()

                    rots = [((lanes + sh) & (LANES - 1)) for sh in range(LANES)]

                    def tr_body(rg, tcarry):
                        rvec = rg * LANES + lanes
                        for dj in range(D // LANES):
                            for sh in range(LANES):
                                dperm = dj * LANES + rots[sh]
                                v = plsc.load_gather(inbs[h], [dperm, rvec])
                                plsc.store_scatter(outbs[h], [rvec, dperm], v)
                        return tcarry
                    lax.fori_loop(0, 128 // LANES, tr_body, 0)

                    @pl.when(nbid < NBLK)
                    def _r():
                        pltpu.async_copy(
                            tt_hbm.at[:, pl.ds(nbid * 128, 128)],
                            inbs[h], rsems[h])
                    pltpu.async_copy(outbs[h],
                                     conv_hbm.at[pl.ds(bid * 128, 128), :],
                                     wsems[h])
            return carry

        lax.fori_loop(0, TRIPS, trip, 0)
        for h in range(2):
            pltpu.make_async_copy(outbs[h], conv_hbm.at[pl.ds(0, 128), :],
                                  wsems[h]).wait()

    return conv_kernel(tt)


def _sc_scores(center, context, negflat, inp, outp):
    mesh = plsc.VectorSubcoreMesh(core_axis_name="c", subcore_axis_name="s")

    @functools.partial(
        pl.kernel,
        mesh=mesh,
        out_type=(jax.ShapeDtypeStruct((B,), jnp.float32),
                  jax.ShapeDtypeStruct((B * NNEG,), jnp.float32)),
        scratch_types=[
            pltpu.VMEM((CHUNK,), jnp.int32),
            pltpu.VMEM((CHUNK,), jnp.int32),
            pltpu.VMEM((CHUNK * NNEG,), jnp.int32),
            pltpu.VMEM((CHUNK, DP), jnp.float32),
            pltpu.VMEM((CHUNK, DP), jnp.float32),
            pltpu.VMEM((CHUNK * NNEG, DP), jnp.float32),
            pltpu.VMEM((CHUNK,), jnp.float32),
            pltpu.VMEM((CHUNK * NNEG,), jnp.float32),
            pltpu.SemaphoreType.DMA,
        ],
        compiler_params=pltpu.CompilerParams(
            needs_layout_passes=False, use_tc_tiling_on_sc=True),
    )
    def sc_kernel(center_hbm, context_hbm, neg_hbm, inemb_hbm, outemb_hbm,
                  pos_hbm, negsc_hbm,
                  cidx_v, oidx_v, nidx_v, crow_v, orow_v, nrow_v,
                  psc_v, nsc_v, sem):
        wid = lax.axis_index("s") * 2 + lax.axis_index("c")
        base = wid * BPW

        def chunk_body(ci, carry):
            start = base + ci * CHUNK
            pltpu.sync_copy(center_hbm.at[pl.ds(start, CHUNK)], cidx_v)
            pltpu.sync_copy(context_hbm.at[pl.ds(start, CHUNK)], oidx_v)
            pltpu.sync_copy(neg_hbm.at[pl.ds(start * NNEG, CHUNK * NNEG)],
                            nidx_v)
            copies = [
                pltpu.async_copy(inemb_hbm.at[cidx_v], crow_v, sem),
                pltpu.async_copy(outemb_hbm.at[oidx_v], orow_v, sem),
            ]
            for j in range(NNEG):
                copies.append(pltpu.async_copy(
                    outemb_hbm.at[nidx_v.at[pl.ds(j * CHUNK, CHUNK)]],
                    nrow_v.at[pl.ds(j * CHUNK, CHUNK)], sem))
            for cp in copies:
                cp.wait()

            def group_body(t, gcarry):
                lanes = lax.iota(jnp.int32, LANES)
                ridx = t * LANES + lanes
                accp = jnp.zeros((LANES,), jnp.float32)
                accn = [jnp.zeros((LANES,), jnp.float32) for _ in range(NNEG)]
                for s in range(D):
                    didx = (lanes + s) & (D - 1)
                    cv = plsc.load_gather(crow_v, [ridx, didx])
                    ov = plsc.load_gather(orow_v, [ridx, didx])
                    accp = accp + cv * ov
                    for k in range(NNEG):
                        nv = plsc.load_gather(
                            nrow_v, [ridx * NNEG + k, didx])
                        accn[k] = accn[k] + cv * nv
                psc_v[pl.ds(t * LANES, LANES)] = accp
                for k in range(NNEG):
                    plsc.store_scatter(nsc_v, [ridx * NNEG + k], accn[k])
                return gcarry

            lax.fori_loop(0, GROUPS, group_body, 0)
            pltpu.sync_copy(psc_v, pos_hbm.at[pl.ds(start, CHUNK)])
            pltpu.sync_copy(nsc_v,
                            negsc_hbm.at[pl.ds(start * NNEG, CHUNK * NNEG)])
            return carry

        lax.fori_loop(0, NCHUNK, chunk_body, 0)

    return sc_kernel(center, context, negflat, inp, outp)


def _tc_loss(pos, neg):
    def body(p_ref, n_ref, o_ref):
        total = jnp.sum(jax.nn.log_sigmoid(p_ref[...]))
        total = total + jnp.sum(jax.nn.log_sigmoid(-n_ref[...]))
        o_ref[...] = jnp.reshape(-total / B, (1, 1))

    return pl.pallas_call(
        body,
        out_shape=jax.ShapeDtypeStruct((1, 1), jnp.float32),
    )(pos, neg)


def kernel(center, context, negatives, in_embed, out_embed):
    center = center.astype(jnp.int32)
    context = context.astype(jnp.int32)
    negflat = negatives.astype(jnp.int32).reshape(B * NNEG)
    inp = _sc_convert(jnp.swapaxes(in_embed, 0, 1))
    outp = _sc_convert(jnp.swapaxes(out_embed, 0, 1))
    pos, neg = _sc_scores(center, context, negflat, inp, outp)
    loss = _tc_loss(pos.reshape(128, B // 128),
                    neg.reshape(1280, B // 128))
    return loss[0, 0]


# speedup vs baseline: 1.2529x; 1.2529x over previous
import functools

import jax
import jax.numpy as jnp
from jax import lax
from jax.experimental import pallas as pl
from jax.experimental.pallas import tpu as pltpu
from jax.experimental.pallas import tpu_sc as plsc

B = 16384
D = 64
DP = 128
NNEG = 10
NW = 32
BPW = B // NW
CHUNK = 16
NCHUNK = BPW // CHUNK
LANES = 16
GROUPS = CHUNK // LANES


def _sc_scores(center, context, negflat, inp, outp):
    mesh = plsc.VectorSubcoreMesh(core_axis_name="c", subcore_axis_name="s")

    set_types = [
        pltpu.VMEM((CHUNK, DP), jnp.float32),
        pltpu.VMEM((CHUNK, DP), jnp.float32),
        pltpu.VMEM((CHUNK * NNEG, DP), jnp.float32),
        pltpu.VMEM((CHUNK,), jnp.float32),
        pltpu.VMEM((CHUNK * NNEG,), jnp.float32),
        pltpu.SemaphoreType.DMA,
    ]
    idx_types = [
        pltpu.VMEM((BPW,), jnp.int32),
        pltpu.VMEM((BPW,), jnp.int32),
        pltpu.VMEM((BPW * NNEG,), jnp.int32),
    ]

    @functools.partial(
        pl.kernel,
        mesh=mesh,
        out_type=(jax.ShapeDtypeStruct((B,), jnp.float32),
                  jax.ShapeDtypeStruct((B * NNEG,), jnp.float32)),
        scratch_types=idx_types + set_types + set_types,
        compiler_params=pltpu.CompilerParams(
            needs_layout_passes=False, use_tc_tiling_on_sc=True),
    )
    def sc_kernel(center_hbm, context_hbm, neg_hbm, inemb_hbm, outemb_hbm,
                  pos_hbm, negsc_hbm, *scratch):
        cidx_v, oidx_v, nidx_v = scratch[:3]
        sets = (scratch[3:9], scratch[9:])
        wid = lax.axis_index("s") * 2 + lax.axis_index("c")
        base = wid * BPW

        pltpu.sync_copy(center_hbm.at[pl.ds(base, BPW)], cidx_v)
        pltpu.sync_copy(context_hbm.at[pl.ds(base, BPW)], oidx_v)
        pltpu.sync_copy(neg_hbm.at[pl.ds(base * NNEG, BPW * NNEG)], nidx_v)

        def fire(ci, st):
            crow_v, orow_v, nrow_v, _, _, sem = st
            lo = ci * CHUNK
            pltpu.async_copy(inemb_hbm.at[cidx_v.at[pl.ds(lo, CHUNK)]],
                             crow_v, sem)
            pltpu.async_copy(outemb_hbm.at[oidx_v.at[pl.ds(lo, CHUNK)]],
                             orow_v, sem)
            for j in range(NNEG):
                pltpu.async_copy(
                    outemb_hbm.at[
                        nidx_v.at[pl.ds(lo * NNEG + j * CHUNK, CHUNK)]],
                    nrow_v.at[pl.ds(j * CHUNK, CHUNK)], sem)

        def drain(ci, st):
            crow_v, orow_v, nrow_v, _, _, sem = st
            lo = ci * CHUNK
            pltpu.make_async_copy(inemb_hbm.at[cidx_v.at[pl.ds(lo, CHUNK)]],
                                  crow_v, sem).wait()
            pltpu.make_async_copy(outemb_hbm.at[oidx_v.at[pl.ds(lo, CHUNK)]],
                                  orow_v, sem).wait()
            for j in range(NNEG):
                pltpu.make_async_copy(
                    outemb_hbm.at[
                        nidx_v.at[pl.ds(lo * NNEG + j * CHUNK, CHUNK)]],
                    nrow_v.at[pl.ds(j * CHUNK, CHUNK)], sem).wait()

        def compute(ci, st):
            crow_v, orow_v, nrow_v, psc_v, nsc_v, _ = st
            start = base + ci * CHUNK

            def group_body(t, gcarry):
                lanes = lax.iota(jnp.int32, LANES)
                ridx = t * LANES + lanes
                accp = jnp.zeros((LANES,), jnp.float32)
                accn = [jnp.zeros((LANES,), jnp.float32) for _ in range(NNEG)]
                for s in range(D):
                    didx = (lanes + s) & (D - 1)
                    cv = plsc.load_gather(crow_v, [ridx, didx])
                    ov = plsc.load_gather(orow_v, [ridx, didx])
                    accp = accp + cv * ov
                    for k in range(NNEG):
                        nv = plsc.load_gather(
                            nrow_v, [ridx * NNEG + k, didx])
                        accn[k] = accn[k] + cv * nv
                psc_v[pl.ds(t * LANES, LANES)] = accp
                for k in range(NNEG):
                    plsc.store_scatter(nsc_v, [ridx * NNEG + k], accn[k])
                return gcarry

            lax.fori_loop(0, GROUPS, group_body, 0)
            pltpu.sync_copy(psc_v, pos_hbm.at[pl.ds(start, CHUNK)])
            pltpu.sync_copy(nsc_v,
                            negsc_hbm.at[pl.ds(start * NNEG, CHUNK * NNEG)])

        fire(0, sets[0])

        def pair_body(t, carry):
            ci0 = 2 * t
            drain(ci0, sets[0])
            fire(ci0 + 1, sets[1])
            compute(ci0, sets[0])
            drain(ci0 + 1, sets[1])

            @pl.when(ci0 + 2 < NCHUNK)
            def _f():
                fire(ci0 + 2, sets[0])
            compute(ci0 + 1, sets[1])
            return carry

        lax.fori_loop(0, NCHUNK // 2, pair_body, 0)

    return sc_kernel(center, context, negflat, inp, outp)


def _tc_loss(pos, neg):
    def body(p_ref, n_ref, o_ref):
        total = jnp.sum(jax.nn.log_sigmoid(p_ref[...]))
        total = total + jnp.sum(jax.nn.log_sigmoid(-n_ref[...]))
        o_ref[...] = jnp.reshape(-total / B, (1, 1))

    return pl.pallas_call(
        body,
        out_shape=jax.ShapeDtypeStruct((1, 1), jnp.float32),
    )(pos, neg)


def kernel(center, context, negatives, in_embed, out_embed):
    center = center.astype(jnp.int32)
    context = context.astype(jnp.int32)
    negflat = negatives.astype(jnp.int32).reshape(B * NNEG)
    inp = jnp.pad(in_embed, ((0, 0), (0, DP - D)))
    outp = jnp.pad(out_embed, ((0, 0), (0, DP - D)))
    pos, neg = _sc_scores(center, context, negflat, inp, outp)
    loss = _tc_loss(pos.reshape(128, B // 128),
                    neg.reshape(1280, B // 128))
    return loss[0, 0]


# speedup vs baseline: 1.2823x; 1.0235x over previous
import functools

import jax
import jax.numpy as jnp
from jax import lax
from jax.experimental import pallas as pl
from jax.experimental.pallas import tpu as pltpu
from jax.experimental.pallas import tpu_sc as plsc

B = 16384
D = 64
DP = 128
NNEG = 10
NW = 32
BPW = B // NW
CHUNK = 64
NCHUNK = BPW // CHUNK
LANES = 16
GROUPS = CHUNK // LANES


def _sc_scores(center, context, negflat, inp, outp):
    mesh = plsc.VectorSubcoreMesh(core_axis_name="c", subcore_axis_name="s")

    @functools.partial(
        pl.kernel,
        mesh=mesh,
        out_type=(jax.ShapeDtypeStruct((B,), jnp.float32),
                  jax.ShapeDtypeStruct((B * NNEG,), jnp.float32)),
        scratch_types=[
            pltpu.VMEM((CHUNK,), jnp.int32),
            pltpu.VMEM((CHUNK,), jnp.int32),
            pltpu.VMEM((CHUNK * NNEG,), jnp.int32),
            pltpu.VMEM((CHUNK, DP), jnp.float32),
            pltpu.VMEM((CHUNK, DP), jnp.float32),
            pltpu.VMEM((CHUNK * NNEG, DP), jnp.float32),
            pltpu.VMEM((CHUNK,), jnp.float32),
            pltpu.VMEM((CHUNK * NNEG,), jnp.float32),
            pltpu.SemaphoreType.DMA,
        ],
        compiler_params=pltpu.CompilerParams(
            needs_layout_passes=False, use_tc_tiling_on_sc=True),
    )
    def sc_kernel(center_hbm, context_hbm, neg_hbm, inemb_hbm, outemb_hbm,
                  pos_hbm, negsc_hbm,
                  cidx_v, oidx_v, nidx_v, crow_v, orow_v, nrow_v,
                  psc_v, nsc_v, sem):
        wid = lax.axis_index("s") * 2 + lax.axis_index("c")
        base = wid * BPW

        def chunk_body(ci, carry):
            start = base + ci * CHUNK
            pltpu.sync_copy(center_hbm.at[pl.ds(start, CHUNK)], cidx_v)
            pltpu.sync_copy(context_hbm.at[pl.ds(start, CHUNK)], oidx_v)
            pltpu.sync_copy(neg_hbm.at[pl.ds(start * NNEG, CHUNK * NNEG)],
                            nidx_v)
            copies = [
                pltpu.async_copy(inemb_hbm.at[cidx_v], crow_v, sem),
                pltpu.async_copy(outemb_hbm.at[oidx_v], orow_v, sem),
            ]
            for j in range(NNEG):
                copies.append(pltpu.async_copy(
                    outemb_hbm.at[nidx_v.at[pl.ds(j * CHUNK, CHUNK)]],
                    nrow_v.at[pl.ds(j * CHUNK, CHUNK)], sem))
            for cp in copies:
                cp.wait()

            def group_body(t, gcarry):
                lanes = lax.iota(jnp.int32, LANES)
                ridx = t * LANES + lanes
                accp = jnp.zeros((LANES,), jnp.float32)
                accn = [jnp.zeros((LANES,), jnp.float32) for _ in range(NNEG)]
                for s in range(D):
                    didx = (lanes + s) & (D - 1)
                    cv = plsc.load_gather(crow_v, [ridx, didx])
                    ov = plsc.load_gather(orow_v, [ridx, didx])
                    accp = accp + cv * ov
                    for k in range(NNEG):
                        nv = plsc.load_gather(
                            nrow_v, [ridx * NNEG + k, didx])
                        accn[k] = accn[k] + cv * nv
                psc_v[pl.ds(t * LANES, LANES)] = accp
                for k in range(NNEG):
                    plsc.store_scatter(nsc_v, [ridx * NNEG + k], accn[k])
                return gcarry

            lax.fori_loop(0, GROUPS, group_body, 0)
            pltpu.sync_copy(psc_v, pos_hbm.at[pl.ds(start, CHUNK)])
            pltpu.sync_copy(nsc_v,
                            negsc_hbm.at[pl.ds(start * NNEG, CHUNK * NNEG)])
            return carry

        lax.fori_loop(0, NCHUNK, chunk_body, 0)

    return sc_kernel(center, context, negflat, inp, outp)


def _tc_loss(pos, neg):
    def body(p_ref, n_ref, o_ref):
        total = jnp.sum(jax.nn.log_sigmoid(p_ref[...]))
        total = total + jnp.sum(jax.nn.log_sigmoid(-n_ref[...]))
        o_ref[...] = jnp.reshape(-total / B, (1, 1))

    return pl.pallas_call(
        body,
        out_shape=jax.ShapeDtypeStruct((1, 1), jnp.float32),
    )(pos, neg)


def kernel(center, context, negatives, in_embed, out_embed):
    center = center.astype(jnp.int32)
    context = context.astype(jnp.int32)
    negflat = negatives.astype(jnp.int32).reshape(B * NNEG)
    inp = jnp.pad(in_embed, ((0, 0), (0, DP - D)))
    outp = jnp.pad(out_embed, ((0, 0), (0, DP - D)))
    pos, neg = _sc_scores(center, context, negflat, inp, outp)
    loss = _tc_loss(pos.reshape(128, B // 128),
                    neg.reshape(1280, B // 128))
    return loss[0, 0]
